# baseline (speedup 1.0000x reference)
import functools

import jax
import jax.numpy as jnp
import numpy as np
from jax import lax
from jax.experimental import pallas as pl
from jax.experimental.pallas import tpu as pltpu
from jax.experimental.pallas import tpu_sc as plsc

REG_LAMBDA = 0.01
LANES = 128
N_CHUNKS = 2


def _gather_body(chunks, woff, h_i, p_i, n_i, r_i, etab, rtab,
                 h_o, p_o, n_o, r_o, idx_v, rows_v, half_v, rel_sp,
                 gsem, osem0, osem1, ssem):
    info = plsc.get_sparse_core_info()
    nc = info.num_cores
    sid = lax.axis_index("s")
    wid = sid * nc + lax.axis_index("c")
    osems = (osem0, osem1)
    state = {"si": 0, "pending": [None, None]}

    @pl.when(sid == 0)
    def _():
        pltpu.async_copy(rtab, rel_sp, ssem)

    rows_per_w = chunks * LANES
    half = LANES // 2

    for i, idx_hbm in enumerate((h_i, p_i, n_i, r_i)):
        pltpu.sync_copy(idx_hbm.at[woff + wid],
                        idx_v.at[pl.ds(i * chunks, chunks)])

    n_stages = 4 * chunks

    def fire(s, buf):
        tab = (etab, etab, etab, rel_sp)[s // chunks]
        return pltpu.async_copy(tab.at[idx_v.at[s]], rows_v.at[buf], gsem)

    def convert(buf):
        bias = jnp.uint32(0x7FFF)
        one = jnp.uint32(1)
        hi_mask = jnp.uint32(0xFFFF0000)

        def body(row, _):
            for g in range(half // 16):
                a = rows_v[buf, row, pl.ds(32 * g, 16)]
                b = rows_v[buf, row, pl.ds(32 * g + 16, 16)]
                ua = lax.bitcast_convert_type(a, jnp.uint32)
                ub = lax.bitcast_convert_type(b, jnp.uint32)
                ua = ua + bias + ((ua >> 16) & one)
                ub = ub + bias + ((ub >> 16) & one)
                w = (ua >> 16) | (ub & hi_mask)
                half_v[buf, row, pl.ds(16 * g, 16)] = (
                    lax.bitcast_convert_type(w, jnp.float32))
            return 0
        lax.fori_loop(0, LANES, body, 0, unroll=False)

    outs = (h_o, p_o, n_o, r_o)
    gath = fire(0, 0)
    for s in range(n_stages):
        buf = s % 2
        gath.wait()
        if s < n_stages - 1:
            if s == n_stages - chunks - 1:
                @pl.when(sid == 0)
                def _():
                    pltpu.make_async_copy(rtab, rel_sp, ssem).wait()
                plsc.subcore_barrier()
            gath = fire(s + 1, 1 - buf)
        if state["pending"][buf] is not None:
            state["pending"][buf].wait()
        convert(buf)
        state["pending"][buf] = pltpu.async_copy(
            half_v.at[buf],
            outs[s // chunks].at[
                pl.ds(wid * rows_per_w + (s % chunks) * LANES, LANES)],
            osems[buf])
    for p in state["pending"]:
        if p is not None:
            p.wait()


def _sc_gather(cidx, h_i, p_i, n_i, r_i, entity_table, relation_table):
    nrows, chunks, _ = h_i.shape
    nw = nrows // N_CHUNKS
    bc = nw * chunks * LANES
    d = entity_table.shape[1]
    mesh = plsc.VectorSubcoreMesh(core_axis_name="c", subcore_axis_name="s")
    row_t = jax.ShapeDtypeStruct((bc, d // 2), jnp.float32)
    kern = functools.partial(
        pl.kernel,
        mesh=mesh,
        out_type=[row_t, row_t, row_t, row_t],
        scratch_types=[
            pltpu.VMEM((4 * chunks, LANES), jnp.int32),
            pltpu.VMEM((2, LANES, d), jnp.float32),
            pltpu.VMEM((2, LANES, d // 2), jnp.float32),
            pltpu.VMEM_SHARED(relation_table.shape, jnp.float32),
            pltpu.SemaphoreType.DMA,
            pltpu.SemaphoreType.DMA,
            pltpu.SemaphoreType.DMA,
            pltpu.SemaphoreType.DMA,
        ],
    )(functools.partial(_gather_body, chunks, cidx * nw))
    return kern(h_i, p_i, n_i, r_i, entity_table, relation_table)


def _loss_body(nb, gh, gp, gn, gr, w_ref, out_ref, acc_ref):
    i = pl.program_id(0)

    @pl.when(i == 0)
    def _():
        acc_ref[0] = 0.0
        acc_ref[1] = 0.0

    w = w_ref[...]
    hf = gh[...].astype(jnp.float32)
    pf = gp[...].astype(jnp.float32)
    nf = gn[...].astype(jnp.float32)
    he = jnp.abs(jnp.dot(hf, w, preferred_element_type=jnp.float32))
    pe = jnp.abs(jnp.dot(pf, w, preferred_element_type=jnp.float32))
    ne = jnp.abs(jnp.dot(nf, w, preferred_element_type=jnp.float32))
    re = jnp.abs(gr[...].astype(jnp.float32))

    base = he + re
    dpos = base - pe
    dneg = base - ne
    pos_s = 0.5 * jnp.sum(dpos * dpos, axis=1, keepdims=True)
    neg_s = 0.5 * jnp.sum(dneg * dneg, axis=1, keepdims=True)
    x = neg_s - pos_s
    logsig = jnp.minimum(x, 0.0) - jnp.log1p(jnp.exp(-jnp.abs(x)))
    sq = (jnp.sum(he * he) + jnp.sum(re * re)
          + jnp.sum(pe * pe) + jnp.sum(ne * ne))
    acc_ref[0] += jnp.sum(logsig)
    acc_ref[1] += sq

    @pl.when(i == nb - 1)
    def _():
        out_ref[0, 0] = acc_ref[0]
        out_ref[0, 1] = acc_ref[1]


def _tc_partial(gh, gp, gn, gr, W):
    bc, d = gh.shape
    bsz = 2048
    nb = bc // bsz
    spec = pl.BlockSpec((bsz, d), lambda i: (i, 0))
    return pl.pallas_call(
        functools.partial(_loss_body, nb),
        grid=(nb,),
        in_specs=[spec, spec, spec, spec,
                  pl.BlockSpec((d, d), lambda i: (0, 0))],
        out_specs=pl.BlockSpec(memory_space=pltpu.SMEM),
        out_shape=jax.ShapeDtypeStruct((1, 2), jnp.float32),
        scratch_shapes=[pltpu.SMEM((2,), jnp.float32)],
    )(gh, gp, gn, gr, W)


def kernel(h, r, pos_t, neg_t, entity_table, relation_table, W):
    b = h.shape[0]
    info = plsc.get_sparse_core_info()
    nw = info.num_cores * info.num_subcores
    bc = b // N_CHUNKS
    chunks = bc // (nw * LANES)

    def shape_idx(x):
        return x.reshape(N_CHUNKS * nw, chunks, LANES).astype(jnp.int32)

    hi, ri, pi, ni = (shape_idx(x) for x in (h, r, pos_t, neg_t))

    def as_bf16(x):
        n = x.shape[0]
        return jax.lax.bitcast_convert_type(x, jnp.bfloat16).reshape(n, -1)

    d = entity_table.shape[1]
    j = np.arange(d)
    perm = (j // 32) * 32 + (j % 2) * 16 + (j % 32) // 2
    w_p = W[perm][:, perm]

    partials = []
    for c in range(N_CHUNKS):
        rows = _sc_gather(c, hi, pi, ni, ri, entity_table, relation_table)
        gh, gp, gn, gr = (as_bf16(x) for x in rows)
        partials.append(_tc_partial(gh, gp, gn, gr, w_p))
    acc = partials[0]
    for p in partials[1:]:
        acc = acc + p
    b_total = jnp.float32(b)
    return (-acc[0, 0] / b_total
            + REG_LAMBDA * acc[0, 1] / (2.0 * b_total))


# speedup vs baseline: 3.7502x; 3.7502x over previous
import functools

import jax
import jax.numpy as jnp
import numpy as np
from jax import lax
from jax.experimental import pallas as pl
from jax.experimental.pallas import tpu as pltpu
from jax.experimental.pallas import tpu_sc as plsc

REG_LAMBDA = 0.01
LANES = 128
N_CHUNKS = 2


def _gather_body(chunks, woff, h_i, p_i, n_i, r_i, etab, rtab,
                 h_o, p_o, n_o, r_o, idx_v, rows_v, half_v, rel_sp,
                 gsem, osem0, osem1, ssem):
    info = plsc.get_sparse_core_info()
    nc = info.num_cores
    sid = lax.axis_index("s")
    wid = sid * nc + lax.axis_index("c")
    osems = (osem0, osem1)
    state = {"si": 0, "pending": [None, None]}

    @pl.when(sid == 0)
    def _():
        pltpu.async_copy(rtab, rel_sp, ssem)

    rows_per_w = chunks * LANES
    half = LANES // 2

    for i, idx_hbm in enumerate((h_i, p_i, n_i, r_i)):
        pltpu.sync_copy(idx_hbm.at[woff + wid],
                        idx_v.at[pl.ds(i * chunks, chunks)])

    n_stages = 4 * chunks

    def fire(s, buf):
        tab = (etab, etab, etab, rel_sp)[s // chunks]
        return pltpu.async_copy(tab.at[idx_v.at[s]], rows_v.at[buf], gsem)

    def convert(buf):
        bias = jnp.uint32(0x7FFF)
        one = jnp.uint32(1)
        hi_mask = jnp.uint32(0xFFFF0000)

        def body(row, _):
            for g in range(half // 16):
                a = rows_v[buf, row, pl.ds(32 * g, 16)]
                b = rows_v[buf, row, pl.ds(32 * g + 16, 16)]
                ua = lax.bitcast_convert_type(a, jnp.uint32)
                ub = lax.bitcast_convert_type(b, jnp.uint32)
                ua = ua + bias + ((ua >> 16) & one)
                ub = ub + bias + ((ub >> 16) & one)
                w = (ua >> 16) | (ub & hi_mask)
                half_v[buf, row, pl.ds(16 * g, 16)] = (
                    lax.bitcast_convert_type(w, jnp.float32))
            return 0
        lax.fori_loop(0, LANES, body, 0, unroll=False)

    outs = (h_o, p_o, n_o, r_o)
    gath = fire(0, 0)
    for s in range(n_stages):
        buf = s % 2
        gath.wait()
        if s < n_stages - 1:
            if s == n_stages - chunks - 1:
                @pl.when(sid == 0)
                def _():
                    pltpu.make_async_copy(rtab, rel_sp, ssem).wait()
                plsc.subcore_barrier()
            gath = fire(s + 1, 1 - buf)
        if state["pending"][buf] is not None:
            state["pending"][buf].wait()
        convert(buf)
        state["pending"][buf] = pltpu.async_copy(
            half_v.at[buf],
            outs[s // chunks].at[
                pl.ds(wid * rows_per_w + (s % chunks) * LANES, LANES)],
            osems[buf])
    for p in state["pending"]:
        if p is not None:
            p.wait()


def _sc_gather(cidx, h_i, p_i, n_i, r_i, entity_table, relation_table):
    nrows, chunks, _ = h_i.shape
    nw = nrows // N_CHUNKS
    bc = nw * chunks * LANES
    d = entity_table.shape[1]
    mesh = plsc.VectorSubcoreMesh(core_axis_name="c", subcore_axis_name="s")
    row_t = jax.ShapeDtypeStruct((bc, d // 2), jnp.float32)
    kern = functools.partial(
        pl.kernel,
        mesh=mesh,
        out_type=[row_t, row_t, row_t, row_t],
        scratch_types=[
            pltpu.VMEM((4 * chunks, LANES), jnp.int32),
            pltpu.VMEM((2, LANES, d), jnp.float32),
            pltpu.VMEM((2, LANES, d // 2), jnp.float32),
            pltpu.VMEM_SHARED(relation_table.shape, jnp.float32),
            pltpu.SemaphoreType.DMA,
            pltpu.SemaphoreType.DMA,
            pltpu.SemaphoreType.DMA,
            pltpu.SemaphoreType.DMA,
        ],
    )(functools.partial(_gather_body, chunks, cidx * nw))
    return kern(h_i, p_i, n_i, r_i, entity_table, relation_table)


def _loss_body(nb, gh, gp, gn, gr, w_ref, out_ref, acc_ref):
    i = pl.program_id(0)

    @pl.when(i == 0)
    def _():
        acc_ref[0] = 0.0
        acc_ref[1] = 0.0

    w = w_ref[...]

    def unpack(ref):
        u = lax.bitcast_convert_type(ref[...], jnp.uint32)
        a = lax.bitcast_convert_type(u << 16, jnp.float32)
        b = lax.bitcast_convert_type(u & jnp.uint32(0xFFFF0000),
                                     jnp.float32)
        return jnp.concatenate([a, b], axis=1)

    he = jnp.abs(jnp.dot(unpack(gh), w, preferred_element_type=jnp.float32))
    pe = jnp.abs(jnp.dot(unpack(gp), w, preferred_element_type=jnp.float32))
    ne = jnp.abs(jnp.dot(unpack(gn), w, preferred_element_type=jnp.float32))
    re = jnp.abs(unpack(gr))

    base = he + re
    dpos = base - pe
    dneg = base - ne
    pos_s = 0.5 * jnp.sum(dpos * dpos, axis=1, keepdims=True)
    neg_s = 0.5 * jnp.sum(dneg * dneg, axis=1, keepdims=True)
    x = neg_s - pos_s
    logsig = jnp.minimum(x, 0.0) - jnp.log1p(jnp.exp(-jnp.abs(x)))
    sq = (jnp.sum(he * he) + jnp.sum(re * re)
          + jnp.sum(pe * pe) + jnp.sum(ne * ne))
    acc_ref[0] += jnp.sum(logsig)
    acc_ref[1] += sq

    @pl.when(i == nb - 1)
    def _():
        out_ref[0, 0] = acc_ref[0]
        out_ref[0, 1] = acc_ref[1]


def _tc_partial(gh, gp, gn, gr, W):
    bc, dh = gh.shape
    d = 2 * dh
    bsz = 2048
    nb = bc // bsz
    spec = pl.BlockSpec((bsz, dh), lambda i: (i, 0))
    return pl.pallas_call(
        functools.partial(_loss_body, nb),
        grid=(nb,),
        in_specs=[spec, spec, spec, spec,
                  pl.BlockSpec((d, d), lambda i: (0, 0))],
        out_specs=pl.BlockSpec(memory_space=pltpu.SMEM),
        out_shape=jax.ShapeDtypeStruct((1, 2), jnp.float32),
        scratch_shapes=[pltpu.SMEM((2,), jnp.float32)],
    )(gh, gp, gn, gr, W)


def kernel(h, r, pos_t, neg_t, entity_table, relation_table, W):
    b = h.shape[0]
    info = plsc.get_sparse_core_info()
    nw = info.num_cores * info.num_subcores
    bc = b // N_CHUNKS
    chunks = bc // (nw * LANES)

    def shape_idx(x):
        return x.reshape(N_CHUNKS * nw, chunks, LANES).astype(jnp.int32)

    hi, ri, pi, ni = (shape_idx(x) for x in (h, r, pos_t, neg_t))

    d = entity_table.shape[1]
    dh = d // 2
    j = np.arange(d)
    jj = j % dh
    perm = 32 * (jj // 16) + 16 * (j // dh) + (jj % 16)
    w_p = W[perm][:, perm]

    partials = []
    for c in range(N_CHUNKS):
        rows = _sc_gather(c, hi, pi, ni, ri, entity_table, relation_table)
        partials.append(_tc_partial(*rows, w_p))
    acc = partials[0]
    for p in partials[1:]:
        acc = acc + p
    b_total = jnp.float32(b)
    return (-acc[0, 0] / b_total
            + REG_LAMBDA * acc[0, 1] / (2.0 * b_total))


# speedup vs baseline: 3.9654x; 1.0574x over previous
import functools

import jax
import jax.numpy as jnp
from jax import lax
from jax.experimental import pallas as pl
from jax.experimental.pallas import tpu as pltpu
from jax.experimental.pallas import tpu_sc as plsc

REG_LAMBDA = 0.01
LANES = 128
N_CHUNKS = 2


def _gather_body(chunks, woff, h_i, p_i, n_i, r_i, etab, rtab,
                 h_o, p_o, n_o, r_o, idx_v, rows_v, rel_sp,
                 gsem, osem0, osem1, ssem):
    info = plsc.get_sparse_core_info()
    nc = info.num_cores
    sid = lax.axis_index("s")
    wid = sid * nc + lax.axis_index("c")
    osems = (osem0, osem1)
    state = {"si": 0, "pending": [None, None]}

    @pl.when(sid == 0)
    def _():
        pltpu.async_copy(rtab, rel_sp, ssem)

    rows_per_w = chunks * LANES

    for i, idx_hbm in enumerate((h_i, p_i, n_i, r_i)):
        pltpu.sync_copy(idx_hbm.at[woff + wid],
                        idx_v.at[pl.ds(i * chunks, chunks)])

    n_stages = 4 * chunks

    def fire(s, buf):
        tab = (etab, etab, etab, rel_sp)[s // chunks]
        return pltpu.async_copy(tab.at[idx_v.at[s]], rows_v.at[buf], gsem)

    outs = (h_o, p_o, n_o, r_o)
    gath = fire(0, 0)
    for s in range(n_stages):
        buf = s % 2
        gath.wait()
        if s < n_stages - 1:
            if s == n_stages - chunks - 1:
                @pl.when(sid == 0)
                def _():
                    pltpu.make_async_copy(rtab, rel_sp, ssem).wait()
                plsc.subcore_barrier()
            if state["pending"][1 - buf] is not None:
                state["pending"][1 - buf].wait()
                state["pending"][1 - buf] = None
            gath = fire(s + 1, 1 - buf)
        if state["pending"][buf] is not None:
            state["pending"][buf].wait()
        state["pending"][buf] = pltpu.async_copy(
            rows_v.at[buf],
            outs[s // chunks].at[
                pl.ds(wid * rows_per_w + (s % chunks) * LANES, LANES)],
            osems[buf])
    for p in state["pending"]:
        if p is not None:
            p.wait()


def _sc_gather(cidx, h_i, p_i, n_i, r_i, entity_table, relation_table):
    nrows, chunks, _ = h_i.shape
    nw = nrows // N_CHUNKS
    bc = nw * chunks * LANES
    d = entity_table.shape[1]
    mesh = plsc.VectorSubcoreMesh(core_axis_name="c", subcore_axis_name="s")
    row_t = jax.ShapeDtypeStruct((bc, d), jnp.float32)
    kern = functools.partial(
        pl.kernel,
        mesh=mesh,
        out_type=[row_t, row_t, row_t, row_t],
        scratch_types=[
            pltpu.VMEM((4 * chunks, LANES), jnp.int32),
            pltpu.VMEM((2, LANES, d), jnp.float32),
            pltpu.VMEM_SHARED(relation_table.shape, jnp.float32),
            pltpu.SemaphoreType.DMA,
            pltpu.SemaphoreType.DMA,
            pltpu.SemaphoreType.DMA,
            pltpu.SemaphoreType.DMA,
        ],
    )(functools.partial(_gather_body, chunks, cidx * nw))
    return kern(h_i, p_i, n_i, r_i, entity_table, relation_table)


def _loss_body(nb, gh, gp, gn, gr, w_ref, out_ref, acc_ref):
    i = pl.program_id(0)

    @pl.when(i == 0)
    def _():
        acc_ref[0] = 0.0
        acc_ref[1] = 0.0

    w = w_ref[...]
    he = jnp.abs(jnp.dot(gh[...], w, preferred_element_type=jnp.float32))
    pe = jnp.abs(jnp.dot(gp[...], w, preferred_element_type=jnp.float32))
    ne = jnp.abs(jnp.dot(gn[...], w, preferred_element_type=jnp.float32))
    re = jnp.abs(gr[...])

    base = he + re
    dpos = base - pe
    dneg = base - ne
    pos_s = 0.5 * jnp.sum(dpos * dpos, axis=1, keepdims=True)
    neg_s = 0.5 * jnp.sum(dneg * dneg, axis=1, keepdims=True)
    x = neg_s - pos_s
    logsig = jnp.minimum(x, 0.0) - jnp.log1p(jnp.exp(-jnp.abs(x)))
    sq = (jnp.sum(he * he) + jnp.sum(re * re)
          + jnp.sum(pe * pe) + jnp.sum(ne * ne))
    acc_ref[0] += jnp.sum(logsig)
    acc_ref[1] += sq

    @pl.when(i == nb - 1)
    def _():
        out_ref[0, 0] = acc_ref[0]
        out_ref[0, 1] = acc_ref[1]


def _tc_partial(gh, gp, gn, gr, W):
    bc, d = gh.shape
    bsz = 2048
    nb = bc // bsz
    spec = pl.BlockSpec((bsz, d), lambda i: (i, 0))
    return pl.pallas_call(
        functools.partial(_loss_body, nb),
        grid=(nb,),
        in_specs=[spec, spec, spec, spec,
                  pl.BlockSpec((d, d), lambda i: (0, 0))],
        out_specs=pl.BlockSpec(memory_space=pltpu.SMEM),
        out_shape=jax.ShapeDtypeStruct((1, 2), jnp.float32),
        scratch_shapes=[pltpu.SMEM((2,), jnp.float32)],
    )(gh, gp, gn, gr, W)


def kernel(h, r, pos_t, neg_t, entity_table, relation_table, W):
    b = h.shape[0]
    info = plsc.get_sparse_core_info()
    nw = info.num_cores * info.num_subcores
    bc = b // N_CHUNKS
    chunks = bc // (nw * LANES)

    def shape_idx(x):
        return x.reshape(N_CHUNKS * nw, chunks, LANES).astype(jnp.int32)

    hi, ri, pi, ni = (shape_idx(x) for x in (h, r, pos_t, neg_t))

    partials = []
    for c in range(N_CHUNKS):
        rows = _sc_gather(c, hi, pi, ni, ri, entity_table, relation_table)
        partials.append(_tc_partial(*rows, W))
    acc = partials[0]
    for p in partials[1:]:
        acc = acc + p
    b_total = jnp.float32(b)
    return (-acc[0, 0] / b_total
            + REG_LAMBDA * acc[0, 1] / (2.0 * b_total))


# speedup vs baseline: 4.1834x; 1.0550x over previous
import functools

import jax
import jax.numpy as jnp
from jax import lax
from jax.experimental import pallas as pl
from jax.experimental.pallas import tpu as pltpu
from jax.experimental.pallas import tpu_sc as plsc

REG_LAMBDA = 0.01
LANES = 128
N_CHUNKS = 2


def _gather_body(chunks, woff, h_i, p_i, n_i, r_i, etab, rtab,
                 h_o, p_o, n_o, r_o, idx_v, rows_v, rel_sp,
                 gsem, osem0, osem1, ssem):
    info = plsc.get_sparse_core_info()
    nc = info.num_cores
    sid = lax.axis_index("s")
    wid = sid * nc + lax.axis_index("c")
    osems = (osem0, osem1)
    state = {"si": 0, "pending": [None, None]}

    @pl.when(sid == 0)
    def _():
        pltpu.async_copy(rtab, rel_sp, ssem)

    rows_per_w = chunks * LANES

    for i, idx_hbm in enumerate((h_i, p_i, n_i, r_i)):
        pltpu.sync_copy(idx_hbm.at[woff + wid],
                        idx_v.at[pl.ds(i * chunks, chunks)])

    n_stages = 4

    def fire(s, buf):
        tab = (etab, etab, etab, rel_sp)[s]
        return [pltpu.async_copy(tab.at[idx_v.at[s * chunks + j]],
                                 rows_v.at[buf, pl.ds(j * LANES, LANES)],
                                 gsem)
                for j in range(chunks)]

    outs = (h_o, p_o, n_o, r_o)
    gath = fire(0, 0)
    for s in range(n_stages):
        buf = s % 2
        for g in gath:
            g.wait()
        if s < n_stages - 1:
            if s == n_stages - 2:
                @pl.when(sid == 0)
                def _():
                    pltpu.make_async_copy(rtab, rel_sp, ssem).wait()
                plsc.subcore_barrier()
            if state["pending"][1 - buf] is not None:
                state["pending"][1 - buf].wait()
                state["pending"][1 - buf] = None
            gath = fire(s + 1, 1 - buf)
        if state["pending"][buf] is not None:
            state["pending"][buf].wait()
        state["pending"][buf] = pltpu.async_copy(
            rows_v.at[buf],
            outs[s].at[pl.ds(wid * rows_per_w, rows_per_w)],
            osems[buf])
    for p in state["pending"]:
        if p is not None:
            p.wait()


def _sc_gather(cidx, h_i, p_i, n_i, r_i, entity_table, relation_table):
    nrows, chunks, _ = h_i.shape
    nw = nrows // N_CHUNKS
    bc = nw * chunks * LANES
    d = entity_table.shape[1]
    mesh = plsc.VectorSubcoreMesh(core_axis_name="c", subcore_axis_name="s")
    row_t = jax.ShapeDtypeStruct((bc, d), jnp.float32)
    kern = functools.partial(
        pl.kernel,
        mesh=mesh,
        out_type=[row_t, row_t, row_t, row_t],
        scratch_types=[
            pltpu.VMEM((4 * chunks, LANES), jnp.int32),
            pltpu.VMEM((2, chunks * LANES, d), jnp.float32),
            pltpu.VMEM_SHARED(relation_table.shape, jnp.float32),
            pltpu.SemaphoreType.DMA,
            pltpu.SemaphoreType.DMA,
            pltpu.SemaphoreType.DMA,
            pltpu.SemaphoreType.DMA,
        ],
    )(functools.partial(_gather_body, chunks, cidx * nw))
    return kern(h_i, p_i, n_i, r_i, entity_table, relation_table)


def _loss_body(nb, gh, gp, gn, gr, w_ref, out_ref, acc_ref):
    i = pl.program_id(0)

    @pl.when(i == 0)
    def _():
        acc_ref[0] = 0.0
        acc_ref[1] = 0.0

    w = w_ref[...]
    he = jnp.abs(jnp.dot(gh[...], w, preferred_element_type=jnp.float32))
    pe = jnp.abs(jnp.dot(gp[...], w, preferred_element_type=jnp.float32))
    ne = jnp.abs(jnp.dot(gn[...], w, preferred_element_type=jnp.float32))
    re = jnp.abs(gr[...])

    base = he + re
    dpos = base - pe
    dneg = base - ne
    pos_s = 0.5 * jnp.sum(dpos * dpos, axis=1, keepdims=True)
    neg_s = 0.5 * jnp.sum(dneg * dneg, axis=1, keepdims=True)
    x = neg_s - pos_s
    logsig = jnp.minimum(x, 0.0) - jnp.log1p(jnp.exp(-jnp.abs(x)))
    sq = (jnp.sum(he * he) + jnp.sum(re * re)
          + jnp.sum(pe * pe) + jnp.sum(ne * ne))
    acc_ref[0] += jnp.sum(logsig)
    acc_ref[1] += sq

    @pl.when(i == nb - 1)
    def _():
        out_ref[0, 0] = acc_ref[0]
        out_ref[0, 1] = acc_ref[1]


def _tc_partial(gh, gp, gn, gr, W):
    bc, d = gh.shape
    bsz = 2048
    nb = bc // bsz
    spec = pl.BlockSpec((bsz, d), lambda i: (i, 0))
    return pl.pallas_call(
        functools.partial(_loss_body, nb),
        grid=(nb,),
        in_specs=[spec, spec, spec, spec,
                  pl.BlockSpec((d, d), lambda i: (0, 0))],
        out_specs=pl.BlockSpec(memory_space=pltpu.SMEM),
        out_shape=jax.ShapeDtypeStruct((1, 2), jnp.float32),
        scratch_shapes=[pltpu.SMEM((2,), jnp.float32)],
    )(gh, gp, gn, gr, W)


def kernel(h, r, pos_t, neg_t, entity_table, relation_table, W):
    b = h.shape[0]
    info = plsc.get_sparse_core_info()
    nw = info.num_cores * info.num_subcores
    bc = b // N_CHUNKS
    chunks = bc // (nw * LANES)

    def shape_idx(x):
        return x.reshape(N_CHUNKS * nw, chunks, LANES).astype(jnp.int32)

    hi, ri, pi, ni = (shape_idx(x) for x in (h, r, pos_t, neg_t))

    partials = []
    for c in range(N_CHUNKS):
        rows = _sc_gather(c, hi, pi, ni, ri, entity_table, relation_table)
        partials.append(_tc_partial(*rows, W))
    acc = partials[0]
    for p in partials[1:]:
        acc = acc + p
    b_total = jnp.float32(b)
    return (-acc[0, 0] / b_total
            + REG_LAMBDA * acc[0, 1] / (2.0 * b_total))
